# baseline (speedup 1.0000x reference)
import functools

import jax
import jax.numpy as jnp
from jax import lax
from jax.experimental import pallas as pl
from jax.experimental.pallas import tpu as pltpu
from jax.experimental.pallas import tpu_sc as plsc

EMBED = 128
LANES = 16
IDX_PAD = 8


def _sc_body(idx_hbm, w_hbm, c_hbm, out_hbm,
             idx_v, row_w, row_c, out_v, sem_w, sem_c):
    pltpu.sync_copy(idx_hbm, idx_v)
    cp_w = pltpu.async_copy(w_hbm.at[idx_v.at[pl.ds(0, 1)]], row_w, sem_w)
    cp_c = pltpu.async_copy(c_hbm.at[idx_v.at[pl.ds(IDX_PAD, 1)]], row_c, sem_c)
    cp_w.wait()
    cp_c.wait()
    acc = row_w[0, pl.ds(0, LANES)] * row_c[0, pl.ds(0, LANES)]
    for j in range(1, EMBED // LANES):
        acc = acc + row_w[0, pl.ds(j * LANES, LANES)] * row_c[0, pl.ds(j * LANES, LANES)]
    s = acc[0]
    for i in range(1, LANES):
        s = s + acc[i]
    v = jnp.full((LANES,), s, jnp.float32)
    out_v[...] = 1.0 / (1.0 + jnp.exp(-v))
    pltpu.sync_copy(out_v, out_hbm)


_sc_call = functools.partial(
    pl.kernel,
    out_type=jax.ShapeDtypeStruct((LANES,), jnp.float32),
    mesh=plsc.VectorSubcoreMesh(
        core_axis_name="c", subcore_axis_name="s", num_cores=1, num_subcores=1),
    scratch_types=[
        pltpu.VMEM((2 * IDX_PAD,), jnp.int32),
        pltpu.VMEM((1, EMBED), jnp.float32),
        pltpu.VMEM((1, EMBED), jnp.float32),
        pltpu.VMEM((LANES,), jnp.float32),
        pltpu.SemaphoreType.DMA,
        pltpu.SemaphoreType.DMA,
    ],
)(_sc_body)


@jax.jit
def kernel(tgt_word, ctx_word, w, c):
    idx = jnp.concatenate([
        jnp.broadcast_to(tgt_word.reshape(1), (IDX_PAD,)),
        jnp.broadcast_to(ctx_word.reshape(1), (IDX_PAD,)),
    ]).astype(jnp.int32)
    out16 = _sc_call(idx, w, c)
    return out16[0]


# speedup vs baseline: 1.0034x; 1.0034x over previous
import functools

import jax
import jax.numpy as jnp
from jax.experimental import pallas as pl
from jax.experimental.pallas import tpu as pltpu
from jax.experimental.pallas import tpu_sc as plsc

EMBED = 128
LANES = 16
IDX_PAD = 8


def _sc_body(idx_hbm, w_hbm, c_hbm, out_hbm,
             idx_v, row_w, row_c, out_v, sem_w, sem_c):
    pltpu.sync_copy(idx_hbm, idx_v)
    cp_w = pltpu.async_copy(w_hbm.at[idx_v.at[pl.ds(0, 1)]], row_w, sem_w)
    cp_c = pltpu.async_copy(c_hbm.at[idx_v.at[pl.ds(IDX_PAD, 1)]], row_c, sem_c)
    cp_w.wait()
    cp_c.wait()
    acc = row_w[0, pl.ds(0, LANES)] * row_c[0, pl.ds(0, LANES)]
    for j in range(1, EMBED // LANES):
        acc = acc + row_w[0, pl.ds(j * LANES, LANES)] * row_c[0, pl.ds(j * LANES, LANES)]
    s = acc[0]
    for i in range(1, LANES):
        s = s + acc[i]
    v = jnp.full((LANES,), s, jnp.float32)
    out_v[...] = 1.0 / (1.0 + jnp.exp(-v))
    pltpu.sync_copy(out_v, out_hbm)


_sc_call = functools.partial(
    pl.kernel,
    out_type=jax.ShapeDtypeStruct((LANES,), jnp.float32),
    mesh=plsc.VectorSubcoreMesh(
        core_axis_name="c", subcore_axis_name="s", num_cores=1, num_subcores=1),
    scratch_types=[
        pltpu.VMEM((2 * IDX_PAD,), jnp.int32),
        pltpu.VMEM((1, EMBED), jnp.float32),
        pltpu.VMEM((1, EMBED), jnp.float32),
        pltpu.VMEM((LANES,), jnp.float32),
        pltpu.SemaphoreType.DMA,
        pltpu.SemaphoreType.DMA,
    ],
)(_sc_body)


@jax.jit
def kernel(tgt_word, ctx_word, w, c):
    idx = jnp.concatenate([
        jnp.broadcast_to(tgt_word.reshape(1), (IDX_PAD,)),
        jnp.broadcast_to(ctx_word.reshape(1), (IDX_PAD,)),
    ]).astype(jnp.int32)
    out16 = _sc_call(idx, w, c)
    return out16[0]
